# Initial kernel scaffold; baseline (speedup 1.0000x reference)
import jax
import jax.numpy as jnp
from jax.experimental import pallas as pl


def kernel(x, edge_index, W_edge, b_edge, W_node, b_node, W_cat1, b_cat1, W_cat2, b_cat2, W_f1, b_f1, gamma, beta, W_f2, b_f2):
    raise NotImplementedError("write your pallas kernel here")


# speedup vs baseline: 3.2675x; 3.2675x over previous
import functools

import jax
import jax.numpy as jnp
from jax import lax
from jax.experimental import pallas as pl
from jax.experimental.pallas import tpu as pltpu
from jax.experimental.pallas import tpu_sc as plsc

N = 10000
E = 320000
D = 128
H = 128
C = 40

NC = 2
NS = 16
NW = NC * NS
CH = 128
EPT = 10240
E_PAD = NW * EPT
N_ACC = 10240
ROWS_PER_TILE = N_ACC // NS
ZR = 128
BLK = 1000
GRID = N // BLK


def _seg_sum_sc(w_aug, src_p, dst_p):
    mesh = plsc.VectorSubcoreMesh(core_axis_name="c", subcore_axis_name="s")

    @functools.partial(
        pl.kernel,
        out_type=jax.ShapeDtypeStruct((NC, N_ACC, H), jnp.float32),
        mesh=mesh,
        scratch_types=[
            pltpu.VMEM((CH,), jnp.int32),
            pltpu.VMEM((CH,), jnp.int32),
            pltpu.VMEM((CH, H), jnp.float32),
            pltpu.VMEM((ZR, H), jnp.float32),
            pltpu.VMEM_SHARED((N_ACC, H), jnp.float32),
            pltpu.SemaphoreType.DMA,
        ],
    )
    def k(w_hbm, src_hbm, dst_hbm, out_hbm, sidx, didx, rows, zbuf, acc, sem):
        c = lax.axis_index("c")
        s = lax.axis_index("s")

        def zero_row(r, _):
            for j in range(H // 16):
                zbuf[r, pl.ds(j * 16, 16)] = jnp.zeros((16,), jnp.float32)
            return 0

        lax.fori_loop(0, ZR, zero_row, 0)
        for b in range(ROWS_PER_TILE // ZR):
            pltpu.sync_copy(zbuf, acc.at[pl.ds(s * ROWS_PER_TILE + b * ZR, ZR)])
        plsc.subcore_barrier()

        ebase = (c * NS + s) * EPT

        def chunk(k_i, _):
            base = ebase + k_i * CH
            pltpu.sync_copy(src_hbm.at[pl.ds(base, CH)], sidx)
            pltpu.sync_copy(dst_hbm.at[pl.ds(base, CH)], didx)
            pltpu.async_copy(w_hbm.at[sidx], rows, sem).wait()
            pltpu.sync_copy(rows, acc.at[didx], add=True)
            return 0

        lax.fori_loop(0, EPT // CH, chunk, 0)
        plsc.subcore_barrier()
        pltpu.sync_copy(
            acc.at[pl.ds(s * ROWS_PER_TILE, ROWS_PER_TILE)],
            out_hbm.at[c, pl.ds(s * ROWS_PER_TILE, ROWS_PER_TILE)],
        )

    return k(w_aug, src_p, dst_p)


def _tc_a(partials, x, w1t, wnt, w2t, wf1t, be, b1, bn, b2, bf1):
    def body(p0_ref, p1_ref, x_ref, w1t_ref, wnt_ref, w2t_ref, wf1t_ref,
             be_ref, b1_ref, bn_ref, b2_ref, bf1_ref, h_ref, st_ref):
        i = pl.program_id(0)
        a = p0_ref[0] + p1_ref[0] + be_ref[...]
        a2 = a + jnp.dot(a, w1t_ref[...], preferred_element_type=jnp.float32) + b1_ref[...]
        xh = jnp.dot(x_ref[...], wnt_ref[...], preferred_element_type=jnp.float32) + bn_ref[...]
        out = a2 + xh + jnp.dot(xh, w2t_ref[...], preferred_element_type=jnp.float32) + b2_ref[...]
        out = jnp.maximum(out, 0.0)
        h1 = jnp.dot(out, wf1t_ref[...], preferred_element_type=jnp.float32) + bf1_ref[...]
        h1 = jnp.maximum(h1, 0.0)
        h_ref[...] = h1
        upd = jnp.concatenate(
            [jnp.sum(h1, axis=0, keepdims=True),
             jnp.sum(h1 * h1, axis=0, keepdims=True),
             jnp.zeros((6, H), jnp.float32)], axis=0)

        @pl.when(i == 0)
        def _():
            st_ref[...] = upd

        @pl.when(i > 0)
        def _():
            st_ref[...] = st_ref[...] + upd

    full = lambda i: (0, 0)
    return pl.pallas_call(
        body,
        grid=(GRID,),
        in_specs=[
            pl.BlockSpec((1, BLK, H), lambda i: (0, i, 0)),
            pl.BlockSpec((1, BLK, H), lambda i: (1, i, 0)),
            pl.BlockSpec((BLK, D), lambda i: (i, 0)),
            pl.BlockSpec((H, H), full),
            pl.BlockSpec((D, H), full),
            pl.BlockSpec((H, H), full),
            pl.BlockSpec((H, H), full),
            pl.BlockSpec((1, H), full),
            pl.BlockSpec((1, H), full),
            pl.BlockSpec((1, H), full),
            pl.BlockSpec((1, H), full),
            pl.BlockSpec((1, H), full),
        ],
        out_specs=[
            pl.BlockSpec((BLK, H), lambda i: (i, 0)),
            pl.BlockSpec((8, H), full),
        ],
        out_shape=[
            jax.ShapeDtypeStruct((N, H), jnp.float32),
            jax.ShapeDtypeStruct((8, H), jnp.float32),
        ],
    )(partials, partials, x, w1t, wnt, w2t, wf1t, be, b1, bn, b2, bf1)


def _tc_b(h, stats, gamma2, beta2, wf2t, bf2):
    def body(h_ref, st_ref, g_ref, b_ref, wf2t_ref, bf2_ref, o_ref):
        st = st_ref[...]
        mean = st[0:1, :] * (1.0 / N)
        var = st[1:2, :] * (1.0 / N) - mean * mean
        scale = lax.rsqrt(var + 1e-5) * g_ref[...]
        hn = (h_ref[...] - mean) * scale + b_ref[...]
        logits = jnp.dot(hn, wf2t_ref[...], preferred_element_type=jnp.float32) + bf2_ref[...]
        m = jnp.max(logits, axis=1, keepdims=True)
        ex = jnp.exp(logits - m)
        lse = jnp.log(jnp.sum(ex, axis=1, keepdims=True))
        o_ref[...] = logits - m - lse

    full = lambda i: (0, 0)
    return pl.pallas_call(
        body,
        grid=(GRID,),
        in_specs=[
            pl.BlockSpec((BLK, H), lambda i: (i, 0)),
            pl.BlockSpec((8, H), full),
            pl.BlockSpec((1, H), full),
            pl.BlockSpec((1, H), full),
            pl.BlockSpec((H, C), full),
            pl.BlockSpec((1, C), full),
        ],
        out_specs=pl.BlockSpec((BLK, C), lambda i: (i, 0)),
        out_shape=jax.ShapeDtypeStruct((N, C), jnp.float32),
    )(h, stats, gamma2, beta2, wf2t, bf2)


def kernel(x, edge_index, W_edge, b_edge, W_node, b_node, W_cat1, b_cat1,
           W_cat2, b_cat2, W_f1, b_f1, gamma, beta, W_f2, b_f2):
    pad = E_PAD - E
    src_p = jnp.concatenate([edge_index[0], jnp.full((pad,), N, jnp.int32)])
    dst_p = jnp.concatenate([edge_index[1], jnp.zeros((pad,), jnp.int32)])
    w_aug = jnp.concatenate([W_edge, jnp.zeros((8, H), jnp.float32)], axis=0)

    partials = _seg_sum_sc(w_aug, src_p, dst_p)

    h, stats = _tc_a(
        partials, x,
        W_cat1.T, W_node.T, W_cat2.T, W_f1.T,
        b_edge.reshape(1, H), b_cat1.reshape(1, H), b_node.reshape(1, H),
        b_cat2.reshape(1, H), b_f1.reshape(1, H),
    )
    return _tc_b(h, stats, gamma.reshape(1, H), beta.reshape(1, H),
                 W_f2.T, b_f2.reshape(1, C))


# speedup vs baseline: 4.1394x; 1.2668x over previous
import functools

import jax
import jax.numpy as jnp
from jax import lax
from jax.experimental import pallas as pl
from jax.experimental.pallas import tpu as pltpu
from jax.experimental.pallas import tpu_sc as plsc

N = 10000
E = 320000
D = 128
H = 128
C = 40

NC = 2
NS = 16
NW = NC * NS
CH = 128
EPT = 10240
E_PAD = NW * EPT
N_ACC = 10240
ROWS_PER_TILE = N_ACC // NS
ZR = 128
BLK = 1000
GRID = N // BLK


NCHUNK = EPT // CH


def _seg_sum_sc(w_aug, src4, dst_p):
    mesh = plsc.VectorSubcoreMesh(core_axis_name="c", subcore_axis_name="s")

    @functools.partial(
        pl.kernel,
        out_type=jax.ShapeDtypeStruct((NC, N_ACC, H), jnp.float32),
        mesh=mesh,
        scratch_types=[
            pltpu.VMEM((NCHUNK, CH), jnp.int32),
            pltpu.VMEM((CH,), jnp.int32),
            pltpu.VMEM((CH,), jnp.int32),
            pltpu.VMEM((CH, H), jnp.float32),
            pltpu.VMEM((CH, H), jnp.float32),
            pltpu.VMEM_SHARED((N_ACC, H), jnp.float32),
            pltpu.SemaphoreType.DMA,
            pltpu.SemaphoreType.DMA,
            pltpu.SemaphoreType.DMA,
            pltpu.SemaphoreType.DMA,
        ],
    )
    def k(w_hbm, src_hbm, dst_hbm, out_hbm, sidx, didx0, didx1, rows0, rows1,
          acc, semd0, semd1, semg0, semg1):
        c = lax.axis_index("c")
        s = lax.axis_index("s")
        didx = (didx0, didx1)
        rows = (rows0, rows1)
        semd = (semd0, semd1)
        semg = (semg0, semg1)

        wid = c * NS + s
        pltpu.sync_copy(src_hbm.at[wid], sidx)

        def zero_row(r, _):
            for j in range(H // 16):
                rows0[r, pl.ds(j * 16, 16)] = jnp.zeros((16,), jnp.float32)
            return 0

        lax.fori_loop(0, CH, zero_row, 0)
        for b in range(ROWS_PER_TILE // ZR):
            pltpu.sync_copy(rows0, acc.at[pl.ds(s * ROWS_PER_TILE + b * ZR, ZR)])
        plsc.subcore_barrier()

        ebase = wid * EPT

        pltpu.async_copy(dst_hbm.at[pl.ds(ebase, CH)], didx0, semd0)
        pltpu.async_copy(dst_hbm.at[pl.ds(ebase + CH, CH)], didx1, semd1)
        pltpu.async_copy(w_hbm.at[sidx.at[0]], rows0, semg0)

        def step(k_i, _):
            b = lax.rem(k_i, 2)
            for bb in range(2):

                @pl.when(b == bb)
                def _():
                    @pl.when(k_i + 1 < NCHUNK)
                    def _():
                        pltpu.async_copy(w_hbm.at[sidx.at[k_i + 1]],
                                         rows[1 - bb], semg[1 - bb])

                    pltpu.make_async_copy(w_hbm.at[pl.ds(0, CH)], rows[bb],
                                          semg[bb]).wait()
                    pltpu.make_async_copy(dst_hbm.at[pl.ds(0, CH)], didx[bb],
                                          semd[bb]).wait()
                    pltpu.sync_copy(rows[bb], acc.at[didx[bb]], add=True)

                    @pl.when(k_i + 2 < NCHUNK)
                    def _():
                        pltpu.async_copy(
                            dst_hbm.at[pl.ds(ebase + (k_i + 2) * CH, CH)],
                            didx[bb], semd[bb])
            return 0

        lax.fori_loop(0, NCHUNK, step, 0)
        plsc.subcore_barrier()
        pltpu.sync_copy(
            acc.at[pl.ds(s * ROWS_PER_TILE, ROWS_PER_TILE)],
            out_hbm.at[c, pl.ds(s * ROWS_PER_TILE, ROWS_PER_TILE)],
        )

    return k(w_aug, src4, dst_p)


def _tc_a(partials, x, w1t, wnt, w2t, wf1t, be, b1, bn, b2, bf1):
    def body(p0_ref, p1_ref, x_ref, w1t_ref, wnt_ref, w2t_ref, wf1t_ref,
             be_ref, b1_ref, bn_ref, b2_ref, bf1_ref, h_ref, st_ref):
        i = pl.program_id(0)
        a = p0_ref[0] + p1_ref[0] + be_ref[...]
        a2 = a + jnp.dot(a, w1t_ref[...], preferred_element_type=jnp.float32) + b1_ref[...]
        xh = jnp.dot(x_ref[...], wnt_ref[...], preferred_element_type=jnp.float32) + bn_ref[...]
        out = a2 + xh + jnp.dot(xh, w2t_ref[...], preferred_element_type=jnp.float32) + b2_ref[...]
        out = jnp.maximum(out, 0.0)
        h1 = jnp.dot(out, wf1t_ref[...], preferred_element_type=jnp.float32) + bf1_ref[...]
        h1 = jnp.maximum(h1, 0.0)
        h_ref[...] = h1
        upd = jnp.concatenate(
            [jnp.sum(h1, axis=0, keepdims=True),
             jnp.sum(h1 * h1, axis=0, keepdims=True),
             jnp.zeros((6, H), jnp.float32)], axis=0)

        @pl.when(i == 0)
        def _():
            st_ref[...] = upd

        @pl.when(i > 0)
        def _():
            st_ref[...] = st_ref[...] + upd

    full = lambda i: (0, 0)
    return pl.pallas_call(
        body,
        grid=(GRID,),
        in_specs=[
            pl.BlockSpec((1, BLK, H), lambda i: (0, i, 0)),
            pl.BlockSpec((1, BLK, H), lambda i: (1, i, 0)),
            pl.BlockSpec((BLK, D), lambda i: (i, 0)),
            pl.BlockSpec((H, H), full),
            pl.BlockSpec((D, H), full),
            pl.BlockSpec((H, H), full),
            pl.BlockSpec((H, H), full),
            pl.BlockSpec((1, H), full),
            pl.BlockSpec((1, H), full),
            pl.BlockSpec((1, H), full),
            pl.BlockSpec((1, H), full),
            pl.BlockSpec((1, H), full),
        ],
        out_specs=[
            pl.BlockSpec((BLK, H), lambda i: (i, 0)),
            pl.BlockSpec((8, H), full),
        ],
        out_shape=[
            jax.ShapeDtypeStruct((N, H), jnp.float32),
            jax.ShapeDtypeStruct((8, H), jnp.float32),
        ],
    )(partials, partials, x, w1t, wnt, w2t, wf1t, be, b1, bn, b2, bf1)


def _tc_b(h, stats, gamma2, beta2, wf2t, bf2):
    def body(h_ref, st_ref, g_ref, b_ref, wf2t_ref, bf2_ref, o_ref):
        st = st_ref[...]
        mean = st[0:1, :] * (1.0 / N)
        var = st[1:2, :] * (1.0 / N) - mean * mean
        scale = lax.rsqrt(var + 1e-5) * g_ref[...]
        hn = (h_ref[...] - mean) * scale + b_ref[...]
        logits = jnp.dot(hn, wf2t_ref[...], preferred_element_type=jnp.float32) + bf2_ref[...]
        m = jnp.max(logits, axis=1, keepdims=True)
        ex = jnp.exp(logits - m)
        lse = jnp.log(jnp.sum(ex, axis=1, keepdims=True))
        o_ref[...] = logits - m - lse

    full = lambda i: (0, 0)
    return pl.pallas_call(
        body,
        grid=(GRID,),
        in_specs=[
            pl.BlockSpec((BLK, H), lambda i: (i, 0)),
            pl.BlockSpec((8, H), full),
            pl.BlockSpec((1, H), full),
            pl.BlockSpec((1, H), full),
            pl.BlockSpec((H, C), full),
            pl.BlockSpec((1, C), full),
        ],
        out_specs=pl.BlockSpec((BLK, C), lambda i: (i, 0)),
        out_shape=jax.ShapeDtypeStruct((N, C), jnp.float32),
    )(h, stats, gamma2, beta2, wf2t, bf2)


def kernel(x, edge_index, W_edge, b_edge, W_node, b_node, W_cat1, b_cat1,
           W_cat2, b_cat2, W_f1, b_f1, gamma, beta, W_f2, b_f2):
    pad = E_PAD - E
    src_p = jnp.concatenate([edge_index[0], jnp.full((pad,), N, jnp.int32)])
    dst_p = jnp.concatenate([edge_index[1], jnp.zeros((pad,), jnp.int32)])
    src4 = src_p.reshape(NW, NCHUNK, CH)
    w_aug = jnp.concatenate([W_edge, jnp.zeros((8, H), jnp.float32)], axis=0)

    partials = _seg_sum_sc(w_aug, src4, dst_p)

    h, stats = _tc_a(
        partials, x,
        W_cat1.T, W_node.T, W_cat2.T, W_f1.T,
        b_edge.reshape(1, H), b_cat1.reshape(1, H), b_node.reshape(1, H),
        b_cat2.reshape(1, H), b_f1.reshape(1, H),
    )
    return _tc_b(h, stats, gamma.reshape(1, H), beta.reshape(1, H),
                 W_f2.T, b_f2.reshape(1, C))


# speedup vs baseline: 11.8159x; 2.8545x over previous
import functools

import jax
import jax.numpy as jnp
from jax import lax
from jax.experimental import pallas as pl
from jax.experimental.pallas import tpu as pltpu
from jax.experimental.pallas import tpu_sc as plsc

N = 10000
E = 320000
D = 128
H = 128
C = 40

NC = 2
NS = 16
NW = NC * NS
CH = 128
EPT = E // NW
NFULL = EPT // CH
TAIL = EPT - NFULL * CH
TAILOFF = NFULL * CH
N_ACC = 10240
ROWS_PER_TILE = N_ACC // NS
ZR = 128
BLK = 1000
GRID = N // BLK


def _seg_sum_sc(W_edge, src_e, dst_e):
    mesh = plsc.VectorSubcoreMesh(core_axis_name="c", subcore_axis_name="s")

    @functools.partial(
        pl.kernel,
        out_type=jax.ShapeDtypeStruct((NC, N_ACC, H), jnp.float32),
        mesh=mesh,
        scratch_types=[
            pltpu.VMEM((EPT,), jnp.int32),
            pltpu.VMEM((CH,), jnp.int32),
            pltpu.VMEM((CH,), jnp.int32),
            pltpu.VMEM((TAIL,), jnp.int32),
            pltpu.VMEM((CH, H), jnp.float32),
            pltpu.VMEM((CH, H), jnp.float32),
            pltpu.VMEM((TAIL, H), jnp.float32),
            pltpu.VMEM_SHARED((N_ACC, H), jnp.float32),
            pltpu.SemaphoreType.DMA,
            pltpu.SemaphoreType.DMA,
            pltpu.SemaphoreType.DMA,
            pltpu.SemaphoreType.DMA,
        ],
    )
    def k(w_hbm, src_hbm, dst_hbm, out_hbm, sidx, didx0, didx1, didxt, rows0,
          rows1, rowst, acc, semd0, semd1, semg0, semg1):
        c = lax.axis_index("c")
        s = lax.axis_index("s")
        didx = (didx0, didx1)
        rows = (rows0, rows1)
        semd = (semd0, semd1)
        semg = (semg0, semg1)

        wid = c * NS + s
        ebase = wid * EPT
        pltpu.sync_copy(src_hbm.at[pl.ds(ebase, EPT)], sidx)

        def zero_row(r, _):
            for j in range(H // 16):
                rows0[r, pl.ds(j * 16, 16)] = jnp.zeros((16,), jnp.float32)
            return 0

        lax.fori_loop(0, CH, zero_row, 0)
        for b in range(ROWS_PER_TILE // ZR):
            pltpu.sync_copy(rows0, acc.at[pl.ds(s * ROWS_PER_TILE + b * ZR, ZR)])
        plsc.subcore_barrier()

        pltpu.async_copy(dst_hbm.at[pl.ds(ebase, CH)], didx0, semd0)
        pltpu.async_copy(dst_hbm.at[pl.ds(ebase + CH, CH)], didx1, semd1)
        pltpu.async_copy(w_hbm.at[sidx.at[pl.ds(0, CH)]], rows0, semg0)

        def step(k_i, _):
            b = lax.rem(k_i, 2)
            for bb in range(2):

                @pl.when(b == bb)
                def _():
                    @pl.when(k_i + 1 < NFULL)
                    def _():
                        pltpu.async_copy(
                            w_hbm.at[sidx.at[pl.ds((k_i + 1) * CH, CH)]],
                            rows[1 - bb], semg[1 - bb])

                    pltpu.make_async_copy(w_hbm.at[pl.ds(0, CH)], rows[bb],
                                          semg[bb]).wait()
                    pltpu.make_async_copy(dst_hbm.at[pl.ds(0, CH)], didx[bb],
                                          semd[bb]).wait()
                    pltpu.sync_copy(rows[bb], acc.at[didx[bb]], add=True)

                    @pl.when(k_i + 2 < NFULL)
                    def _():
                        pltpu.async_copy(
                            dst_hbm.at[pl.ds(ebase + (k_i + 2) * CH, CH)],
                            didx[bb], semd[bb])
            return 0

        lax.fori_loop(0, NFULL, step, 0)

        pltpu.sync_copy(dst_hbm.at[pl.ds(ebase + TAILOFF, TAIL)], didxt)
        pltpu.async_copy(w_hbm.at[sidx.at[pl.ds(TAILOFF, TAIL)]], rowst,
                         semg0).wait()
        pltpu.sync_copy(rowst, acc.at[didxt], add=True)

        plsc.subcore_barrier()
        pltpu.sync_copy(
            acc.at[pl.ds(s * ROWS_PER_TILE, ROWS_PER_TILE)],
            out_hbm.at[c, pl.ds(s * ROWS_PER_TILE, ROWS_PER_TILE)],
        )

    return k(W_edge, src_e, dst_e)


def _tc_a(partials, x, w1t, wnt, w2t, wf1t, be, b1, bn, b2, bf1):
    def body(p0_ref, p1_ref, x_ref, w1t_ref, wnt_ref, w2t_ref, wf1t_ref,
             be_ref, b1_ref, bn_ref, b2_ref, bf1_ref, h_ref, st_ref):
        i = pl.program_id(0)
        a = p0_ref[0] + p1_ref[0] + be_ref[...]
        a2 = a + jnp.dot(a, w1t_ref[...], preferred_element_type=jnp.float32) + b1_ref[...]
        xh = jnp.dot(x_ref[...], wnt_ref[...], preferred_element_type=jnp.float32) + bn_ref[...]
        out = a2 + xh + jnp.dot(xh, w2t_ref[...], preferred_element_type=jnp.float32) + b2_ref[...]
        out = jnp.maximum(out, 0.0)
        h1 = jnp.dot(out, wf1t_ref[...], preferred_element_type=jnp.float32) + bf1_ref[...]
        h1 = jnp.maximum(h1, 0.0)
        h_ref[...] = h1
        upd = jnp.concatenate(
            [jnp.sum(h1, axis=0, keepdims=True),
             jnp.sum(h1 * h1, axis=0, keepdims=True),
             jnp.zeros((6, H), jnp.float32)], axis=0)

        @pl.when(i == 0)
        def _():
            st_ref[...] = upd

        @pl.when(i > 0)
        def _():
            st_ref[...] = st_ref[...] + upd

    full = lambda i: (0, 0)
    return pl.pallas_call(
        body,
        grid=(GRID,),
        in_specs=[
            pl.BlockSpec((1, BLK, H), lambda i: (0, i, 0)),
            pl.BlockSpec((1, BLK, H), lambda i: (1, i, 0)),
            pl.BlockSpec((BLK, D), lambda i: (i, 0)),
            pl.BlockSpec((H, H), full),
            pl.BlockSpec((D, H), full),
            pl.BlockSpec((H, H), full),
            pl.BlockSpec((H, H), full),
            pl.BlockSpec((1, H), full),
            pl.BlockSpec((1, H), full),
            pl.BlockSpec((1, H), full),
            pl.BlockSpec((1, H), full),
            pl.BlockSpec((1, H), full),
        ],
        out_specs=[
            pl.BlockSpec((BLK, H), lambda i: (i, 0)),
            pl.BlockSpec((8, H), full),
        ],
        out_shape=[
            jax.ShapeDtypeStruct((N, H), jnp.float32),
            jax.ShapeDtypeStruct((8, H), jnp.float32),
        ],
    )(partials, partials, x, w1t, wnt, w2t, wf1t, be, b1, bn, b2, bf1)


def _tc_b(h, stats, gamma2, beta2, wf2t, bf2):
    def body(h_ref, st_ref, g_ref, b_ref, wf2t_ref, bf2_ref, o_ref):
        st = st_ref[...]
        mean = st[0:1, :] * (1.0 / N)
        var = st[1:2, :] * (1.0 / N) - mean * mean
        scale = lax.rsqrt(var + 1e-5) * g_ref[...]
        hn = (h_ref[...] - mean) * scale + b_ref[...]
        logits = jnp.dot(hn, wf2t_ref[...], preferred_element_type=jnp.float32) + bf2_ref[...]
        m = jnp.max(logits, axis=1, keepdims=True)
        ex = jnp.exp(logits - m)
        lse = jnp.log(jnp.sum(ex, axis=1, keepdims=True))
        o_ref[...] = logits - m - lse

    full = lambda i: (0, 0)
    return pl.pallas_call(
        body,
        grid=(GRID,),
        in_specs=[
            pl.BlockSpec((BLK, H), lambda i: (i, 0)),
            pl.BlockSpec((8, H), full),
            pl.BlockSpec((1, H), full),
            pl.BlockSpec((1, H), full),
            pl.BlockSpec((H, C), full),
            pl.BlockSpec((1, C), full),
        ],
        out_specs=pl.BlockSpec((BLK, C), lambda i: (i, 0)),
        out_shape=jax.ShapeDtypeStruct((N, C), jnp.float32),
    )(h, stats, gamma2, beta2, wf2t, bf2)


def kernel(x, edge_index, W_edge, b_edge, W_node, b_node, W_cat1, b_cat1,
           W_cat2, b_cat2, W_f1, b_f1, gamma, beta, W_f2, b_f2):
    partials = _seg_sum_sc(W_edge, edge_index[0], edge_index[1])

    h, stats = _tc_a(
        partials, x,
        W_cat1.T, W_node.T, W_cat2.T, W_f1.T,
        b_edge.reshape(1, H), b_cat1.reshape(1, H), b_node.reshape(1, H),
        b_cat2.reshape(1, H), b_f1.reshape(1, H),
    )
    return _tc_b(h, stats, gamma.reshape(1, H), beta.reshape(1, H),
                 W_f2.T, b_f2.reshape(1, C))
